# Initial kernel scaffold; baseline (speedup 1.0000x reference)
import jax
import jax.numpy as jnp
from jax.experimental import pallas as pl


def kernel(pred_field, tgt_field, matched_pred_birth, matched_pred_death, matched_tgt_birth, matched_tgt_death, unmatched_pred_birth, unmatched_pred_death, unmatched_tgt_birth, unmatched_tgt_death):
    raise NotImplementedError("write your pallas kernel here")


# speedup vs baseline: 1.5001x; 1.5001x over previous
import functools

import jax
import jax.numpy as jnp
from jax import lax
from jax.experimental import pallas as pl
from jax.experimental.pallas import tpu as pltpu
from jax.experimental.pallas import tpu_sc as plsc

NC = 2
NS = 16
NW = NC * NS
L = 16

NM, NU = 20000, 5000
NM_PAD, NU_PAD = 20480, 8192
CM, CU = NM_PAD // NW, NU_PAD // NW
VM, VU = CM // L, CU // L

_F = jnp.float32
_I = jnp.int32


def _build():
  mesh = plsc.VectorSubcoreMesh(
      core_axis_name="c", subcore_axis_name="s",
      num_cores=NC, num_subcores=NS)

  coord_scratch = [pltpu.VMEM((3, CM), _I)] * 4 + [pltpu.VMEM((3, CU), _I)] * 4
  idx_scratch = [pltpu.VMEM((CM,), _I)] * 4 + [pltpu.VMEM((CU,), _I)] * 4
  val_scratch = [pltpu.VMEM((CM,), _F)] * 4 + [pltpu.VMEM((CU,), _F)] * 4

  @functools.partial(
      pl.kernel,
      out_type=jax.ShapeDtypeStruct((NW, L), _F),
      mesh=mesh,
      scratch_types=[*coord_scratch, *idx_scratch, *val_scratch,
                     pltpu.VMEM((L,), _F), pltpu.SemaphoreType.DMA],
  )
  def run(pred_hbm, tgt_hbm,
          mpb, mpd, mtb, mtd, upb, upd, utb, utd,
          out_hbm,
          cv0, cv1, cv2, cv3, cv4, cv5, cv6, cv7,
          iv0, iv1, iv2, iv3, iv4, iv5, iv6, iv7,
          vv0, vv1, vv2, vv3, vv4, vv5, vv6, vv7,
          acc_v, sem):
    wid = lax.axis_index("s") * NC + lax.axis_index("c")
    lanes = lax.iota(_I, L)

    coord_hbms = [mpb, mpd, mtb, mtd, upb, upd, utb, utd]
    coord_vs = [cv0, cv1, cv2, cv3, cv4, cv5, cv6, cv7]
    idx_vs = [iv0, iv1, iv2, iv3, iv4, iv5, iv6, iv7]
    val_vs = [vv0, vv1, vv2, vv3, vv4, vv5, vv6, vv7]
    chunks = [CM] * 4 + [CU] * 4
    tables = [pred_hbm, pred_hbm, tgt_hbm, tgt_hbm,
              pred_hbm, pred_hbm, tgt_hbm, tgt_hbm]
    splits = [(128, CM // 128)] * 4 + [(128, CU // 128)] * 4

    cps = [pltpu.async_copy(hbm.at[:, pl.ds(wid * ch, ch)], cv, sem)
           for hbm, cv, ch in zip(coord_hbms, coord_vs, chunks)]
    for cp in cps:
      cp.wait()

    def linearize(cv, iv, nvec):
      def body(j, carry):
        sl = pl.ds(j * L, L)
        iv[sl] = cv[0, sl] * 16384 + cv[1, sl] * 128 + cv[2, sl]
        return carry
      lax.fori_loop(0, nvec, body, 0)

    for cv, iv, ch in zip(coord_vs, idx_vs, chunks):
      linearize(cv, iv, ch // L)

    gps = []
    for tab, iv, vv, (gs, gn) in zip(tables, idx_vs, val_vs, splits):
      for k in range(gn):
        sl = pl.ds(k * gs, gs)
        gps.append(pltpu.async_copy(tab.at[iv.at[sl]], vv.at[sl], sem))
    for g in gps:
      g.wait()

    def term(va, vb, nvec, ch, n_real):
      base = wid * ch
      def body(j, acc):
        a = va[pl.ds(j * L, L)]
        b = vb[pl.ds(j * L, L)]
        d = a - b
        pos = base + j * L + lanes
        return acc + jnp.where(pos < n_real, d * d, jnp.zeros_like(d))
      return lax.fori_loop(0, nvec, body, jnp.zeros((L,), _F))

    t_b = term(val_vs[0], val_vs[2], VM, CM, NM)
    t_d = term(val_vs[1], val_vs[3], VM, CM, NM)
    t_up = term(val_vs[4], val_vs[5], VU, CU, NU)
    t_ut = term(val_vs[6], val_vs[7], VU, CU, NU)
    acc_v[...] = 2.0 * (t_b + t_d) + (t_up + t_ut)
    pltpu.sync_copy(acc_v, out_hbm.at[wid])

  return run


_run = _build()


def _prep(c, npad):
  return jnp.pad(c, ((0, npad - c.shape[0]), (0, 0))).T


def kernel(pred_field, tgt_field,
           matched_pred_birth, matched_pred_death,
           matched_tgt_birth, matched_tgt_death,
           unmatched_pred_birth, unmatched_pred_death,
           unmatched_tgt_birth, unmatched_tgt_death):
  out = _run(
      pred_field.reshape(-1), tgt_field.reshape(-1),
      _prep(matched_pred_birth, NM_PAD), _prep(matched_pred_death, NM_PAD),
      _prep(matched_tgt_birth, NM_PAD), _prep(matched_tgt_death, NM_PAD),
      _prep(unmatched_pred_birth, NU_PAD), _prep(unmatched_pred_death, NU_PAD),
      _prep(unmatched_tgt_birth, NU_PAD), _prep(unmatched_tgt_death, NU_PAD))
  return jnp.sum(out).reshape(1)


# speedup vs baseline: 1.5635x; 1.0423x over previous
import functools

import jax
import jax.numpy as jnp
from jax import lax
from jax.experimental import pallas as pl
from jax.experimental.pallas import tpu as pltpu
from jax.experimental.pallas import tpu_sc as plsc

NC = 2
NS = 16
NW = NC * NS
L = 16

NM, NU = 20000, 5000
NM_PAD, NU_PAD = 20480, 8192
CM, CU = NM_PAD // NW, NU_PAD // NW
VM, VU = CM // L, CU // L

_F = jnp.float32
_I = jnp.int32


def _build():
  mesh = plsc.VectorSubcoreMesh(
      core_axis_name="c", subcore_axis_name="s",
      num_cores=NC, num_subcores=NS)

  coord_scratch = [pltpu.VMEM((3, CM), _I)] * 4 + [pltpu.VMEM((3, CU), _I)] * 4
  idx_scratch = [pltpu.VMEM((CM,), _I)] * 4 + [pltpu.VMEM((CU,), _I)] * 4
  val_scratch = [pltpu.VMEM((CM,), _F)] * 4 + [pltpu.VMEM((CU,), _F)] * 4

  @functools.partial(
      pl.kernel,
      out_type=jax.ShapeDtypeStruct((NW, L), _F),
      mesh=mesh,
      scratch_types=[*coord_scratch, *idx_scratch, *val_scratch,
                     pltpu.VMEM((L,), _F), pltpu.SemaphoreType.DMA],
  )
  def run(pred_hbm, tgt_hbm,
          mpb, mpd, mtb, mtd, upb, upd, utb, utd,
          out_hbm,
          cv0, cv1, cv2, cv3, cv4, cv5, cv6, cv7,
          iv0, iv1, iv2, iv3, iv4, iv5, iv6, iv7,
          vv0, vv1, vv2, vv3, vv4, vv5, vv6, vv7,
          acc_v, sem):
    wid = lax.axis_index("s") * NC + lax.axis_index("c")
    lanes = lax.iota(_I, L)

    coord_hbms = [mpb, mpd, mtb, mtd, upb, upd, utb, utd]
    coord_vs = [cv0, cv1, cv2, cv3, cv4, cv5, cv6, cv7]
    idx_vs = [iv0, iv1, iv2, iv3, iv4, iv5, iv6, iv7]
    val_vs = [vv0, vv1, vv2, vv3, vv4, vv5, vv6, vv7]
    chunks = [CM] * 4 + [CU] * 4
    tables = [pred_hbm, pred_hbm, tgt_hbm, tgt_hbm,
              pred_hbm, pred_hbm, tgt_hbm, tgt_hbm]
    cps = [pltpu.async_copy(hbm.at[:, pl.ds(wid * ch, ch)], cv, sem)
           for hbm, cv, ch in zip(coord_hbms, coord_vs, chunks)]

    def linearize(cv, iv, nvec):
      def body(j, carry):
        sl = pl.ds(j * L, L)
        iv[sl] = cv[0, sl] * 16384 + cv[1, sl] * 128 + cv[2, sl]
        return carry
      lax.fori_loop(0, nvec, body, 0)

    gps = []
    for cp, tab, cv, iv, vv, ch in zip(
        cps, tables, coord_vs, idx_vs, val_vs, chunks):
      cp.wait()
      linearize(cv, iv, ch // L)
      gps.append(pltpu.async_copy(tab.at[iv], vv, sem))
    for g in gps:
      g.wait()

    def term(va, vb, nvec, ch, n_real):
      base = wid * ch
      def body(j, acc):
        a = va[pl.ds(j * L, L)]
        b = vb[pl.ds(j * L, L)]
        d = a - b
        pos = base + j * L + lanes
        return acc + jnp.where(pos < n_real, d * d, jnp.zeros_like(d))
      return lax.fori_loop(0, nvec, body, jnp.zeros((L,), _F))

    t_b = term(val_vs[0], val_vs[2], VM, CM, NM)
    t_d = term(val_vs[1], val_vs[3], VM, CM, NM)
    t_up = term(val_vs[4], val_vs[5], VU, CU, NU)
    t_ut = term(val_vs[6], val_vs[7], VU, CU, NU)
    acc_v[...] = 2.0 * (t_b + t_d) + (t_up + t_ut)
    pltpu.sync_copy(acc_v, out_hbm.at[wid])

  return run


_run = _build()


def _prep(c, npad):
  return jnp.pad(c, ((0, npad - c.shape[0]), (0, 0))).T


def kernel(pred_field, tgt_field,
           matched_pred_birth, matched_pred_death,
           matched_tgt_birth, matched_tgt_death,
           unmatched_pred_birth, unmatched_pred_death,
           unmatched_tgt_birth, unmatched_tgt_death):
  out = _run(
      pred_field.reshape(-1), tgt_field.reshape(-1),
      _prep(matched_pred_birth, NM_PAD), _prep(matched_pred_death, NM_PAD),
      _prep(matched_tgt_birth, NM_PAD), _prep(matched_tgt_death, NM_PAD),
      _prep(unmatched_pred_birth, NU_PAD), _prep(unmatched_pred_death, NU_PAD),
      _prep(unmatched_tgt_birth, NU_PAD), _prep(unmatched_tgt_death, NU_PAD))
  return jnp.sum(out).reshape(1)


# speedup vs baseline: 3.0652x; 1.9605x over previous
import functools

import jax
import jax.numpy as jnp
from jax import lax
from jax.experimental import pallas as pl
from jax.experimental.pallas import tpu as pltpu
from jax.experimental.pallas import tpu_sc as plsc

NC = 2
NS = 16
NW = NC * NS
L = 16

NM, NU = 20000, 5000
NM_PAD, NU_PAD = 20480, 5120
CM, CU = NM_PAD // NW, NU_PAD // NW
VM, VU = CM // L, CU // L
CM_RUN, CU_RUN = 4 * 3 * CM, 4 * 3 * CU

_F = jnp.float32
_I = jnp.int32


def _build():
  mesh = plsc.VectorSubcoreMesh(
      core_axis_name="c", subcore_axis_name="s",
      num_cores=NC, num_subcores=NS)

  idx_scratch = [pltpu.VMEM((CM,), _I)] * 4 + [pltpu.VMEM((CU,), _I)] * 4
  val_scratch = [pltpu.VMEM((CM,), _F)] * 4 + [pltpu.VMEM((CU,), _F)] * 4

  @functools.partial(
      pl.kernel,
      out_type=jax.ShapeDtypeStruct((NW, L), _F),
      mesh=mesh,
      scratch_types=[pltpu.VMEM((CM_RUN,), _I), pltpu.VMEM((CU_RUN,), _I),
                     *idx_scratch, *val_scratch,
                     pltpu.VMEM((L,), _F), pltpu.SemaphoreType.DMA],
  )
  def run(pred_hbm, tgt_hbm, cm_hbm, cu_hbm,
          out_hbm,
          cvm, cvu,
          iv0, iv1, iv2, iv3, iv4, iv5, iv6, iv7,
          vv0, vv1, vv2, vv3, vv4, vv5, vv6, vv7,
          acc_v, sem):
    wid = lax.axis_index("s") * NC + lax.axis_index("c")
    lanes = lax.iota(_I, L)

    idx_vs = [iv0, iv1, iv2, iv3, iv4, iv5, iv6, iv7]
    val_vs = [vv0, vv1, vv2, vv3, vv4, vv5, vv6, vv7]
    tables = [pred_hbm, pred_hbm, tgt_hbm, tgt_hbm,
              pred_hbm, pred_hbm, tgt_hbm, tgt_hbm]

    cpm = pltpu.async_copy(cm_hbm.at[pl.ds(wid * CM_RUN, CM_RUN)], cvm, sem)
    cpu = pltpu.async_copy(cu_hbm.at[pl.ds(wid * CU_RUN, CU_RUN)], cvu, sem)

    def linearize(cv, base, ch, iv):
      def body(j, carry):
        o = j * L
        iv[pl.ds(o, L)] = (cv[pl.ds(base + o, L)] * 16384
                           + cv[pl.ds(base + ch + o, L)] * 128
                           + cv[pl.ds(base + 2 * ch + o, L)])
        return carry
      lax.fori_loop(0, ch // L, body, 0, unroll=4)

    gps = []
    cpm.wait()
    for l in range(4):
      linearize(cvm, l * 3 * CM, CM, idx_vs[l])
      gps.append(pltpu.async_copy(tables[l].at[idx_vs[l]], val_vs[l], sem))
    cpu.wait()
    for l in range(4):
      linearize(cvu, l * 3 * CU, CU, idx_vs[4 + l])
      gps.append(
          pltpu.async_copy(tables[4 + l].at[idx_vs[4 + l]], val_vs[4 + l], sem))
    for g in gps:
      g.wait()

    def term(va, vb, nvec, ch, n_real):
      base = wid * ch
      def body(j, acc):
        sl = pl.ds(j * L, L)
        d = va[sl] - vb[sl]
        pos = base + j * L + lanes
        return acc + jnp.where(pos < n_real, d * d, jnp.zeros_like(d))
      return lax.fori_loop(0, nvec, body, jnp.zeros((L,), _F), unroll=4)

    t_b = term(val_vs[0], val_vs[2], VM, CM, NM)
    t_d = term(val_vs[1], val_vs[3], VM, CM, NM)
    t_up = term(val_vs[4], val_vs[5], VU, CU, NU)
    t_ut = term(val_vs[6], val_vs[7], VU, CU, NU)
    acc_v[...] = 2.0 * (t_b + t_d) + (t_up + t_ut)
    pltpu.sync_copy(acc_v, out_hbm.at[wid])

  return run


_run = _build()


def _pack(lists, npad):
  ch = npad // NW
  a = jnp.stack([jnp.pad(c, ((0, npad - c.shape[0]), (0, 0))) for c in lists])
  a = a.transpose(0, 2, 1).reshape(4, 3, NW, ch).transpose(2, 0, 1, 3)
  return a.reshape(-1)


def kernel(pred_field, tgt_field,
           matched_pred_birth, matched_pred_death,
           matched_tgt_birth, matched_tgt_death,
           unmatched_pred_birth, unmatched_pred_death,
           unmatched_tgt_birth, unmatched_tgt_death):
  out = _run(
      pred_field.reshape(-1), tgt_field.reshape(-1),
      _pack([matched_pred_birth, matched_pred_death,
             matched_tgt_birth, matched_tgt_death], NM_PAD),
      _pack([unmatched_pred_birth, unmatched_pred_death,
             unmatched_tgt_birth, unmatched_tgt_death], NU_PAD))
  return jnp.sum(out).reshape(1)
